# Initial kernel scaffold; baseline (speedup 1.0000x reference)
import jax
import jax.numpy as jnp
from jax.experimental import pallas as pl


def kernel(inputs):
    raise NotImplementedError("write your pallas kernel here")


# speedup vs baseline: 5.9654x; 5.9654x over previous
import functools

import jax
import jax.numpy as jnp
from jax.experimental import pallas as pl
from jax.experimental.pallas import tpu as pltpu

OLD_CL = 16
RATIO = 0.2
BH = 64


def _loss_kernel(x_ref, out_ref, sq_ref, cnt_ref, *, n_img, n_j, c, h, w):
    i = pl.program_id(0)
    j = pl.program_id(1)

    @pl.when((i == 0) & (j == 0))
    def _init():
        sq_ref[:, :] = jnp.zeros_like(sq_ref)
        cnt_ref[:, :] = jnp.zeros_like(cnt_ref)

    x = x_ref[0]

    m = x[0]
    a = jnp.zeros(m.shape, dtype=jnp.int32)
    for ci in range(1, c):
        upd = x[ci] > m
        m = jnp.where(upd, x[ci], m)
        a = jnp.where(upd, ci, a)

    z = jnp.zeros_like(m)
    s_old = jnp.zeros_like(m)
    e_new = []
    for ci in range(c):
        e = jnp.exp(x[ci] - m)
        z = z + e
        if ci < OLD_CL:
            s_old = s_old + e
        else:
            e_new.append(e)
    inv_z = 1.0 / z

    row = jax.lax.broadcasted_iota(jnp.int32, (8, 128), 0)
    lane = jax.lax.broadcasted_iota(jnp.int32, (8, 128), 1)

    sq_upd = jnp.zeros((8, 128), dtype=jnp.float32)
    cnt_upd = jnp.zeros((8, 128), dtype=jnp.float32)

    p0 = s_old * inv_z
    sq_vals = [jnp.sum(p0 * p0)]
    for e in e_new:
        p = e * inv_z
        sq_vals.append(jnp.sum(p * p))
    cnt_vals = [jnp.sum(jnp.where(a < OLD_CL, 1.0, 0.0))]
    for ci in range(OLD_CL, c):
        cnt_vals.append(jnp.sum(jnp.where(a == ci, 1.0, 0.0)))

    for k in range(len(sq_vals)):
        mask = (row == i) & (lane == k)
        sq_upd = sq_upd + jnp.where(mask, sq_vals[k], 0.0)
        cnt_upd = cnt_upd + jnp.where(mask, cnt_vals[k], 0.0)

    sq_ref[:, :] = sq_ref[:, :] + sq_upd
    cnt_ref[:, :] = cnt_ref[:, :] + cnt_upd

    @pl.when((i == n_img - 1) & (j == n_j - 1))
    def _finish():
        nbin = c - OLD_CL + 1
        valid = (row < n_img) & (lane < nbin)
        cnt = cnt_ref[:, :]
        safe = jnp.where(valid, jnp.where(cnt == 0.0, 1.0, cnt), 1.0)
        total = jnp.sum(jnp.where(valid, safe, 0.0), axis=1, keepdims=True)
        wgt = jnp.where(valid, jnp.power(total / safe, RATIO), 0.0)
        contrib = jnp.sum(sq_ref[:, :] * wgt)
        out_ref[0, 0] = -contrib / (n_img * c * h * w)


def kernel(inputs):
    n, c, h, w = inputs.shape
    n_j = h // BH
    out = pl.pallas_call(
        functools.partial(_loss_kernel, n_img=n, n_j=n_j, c=c, h=h, w=w),
        grid=(n, n_j),
        in_specs=[
            pl.BlockSpec((1, c, BH, w), lambda i, j: (i, 0, j, 0)),
        ],
        out_specs=pl.BlockSpec(
            (1, 1), lambda i, j: (0, 0), memory_space=pltpu.SMEM
        ),
        out_shape=jax.ShapeDtypeStruct((1, 1), jnp.float32),
        scratch_shapes=[
            pltpu.VMEM((8, 128), jnp.float32),
            pltpu.VMEM((8, 128), jnp.float32),
        ],
    )(inputs)
    return out[0, 0]
